# Initial kernel scaffold; baseline (speedup 1.0000x reference)
import jax
import jax.numpy as jnp
from jax.experimental import pallas as pl


def kernel(x, edge_index, W1, b1, W2, b2, W3, b3, Wl, bl):
    raise NotImplementedError("write your pallas kernel here")


# speedup vs baseline: 12.8739x; 12.8739x over previous
import functools

import jax
import jax.numpy as jnp
from jax import lax
from jax.experimental import pallas as pl
from jax.experimental.pallas import tpu as pltpu
from jax.experimental.pallas import tpu_sc as plsc

N = 10000
D = 128
C = 10

NC = 2
NS = 16
NW = NC * NS

K = 128
E = 320000
EW = 10112
IT = EW // K
E_P = EW * NW
N_P = 10240
RPT = N_P // NS

RB = 1280
TGRID = N_P // RB


def _sc_mesh():
    return plsc.VectorSubcoreMesh(core_axis_name="c", subcore_axis_name="s")


NR = N_P // 128


@functools.partial(
    pl.kernel,
    out_type=jax.ShapeDtypeStruct((NW, NR, 128), jnp.float32),
    mesh=_sc_mesh(),
    compiler_params=pltpu.CompilerParams(needs_layout_passes=False),
    scratch_types=[
        pltpu.VMEM((K,), jnp.int32),
        pltpu.VMEM((NR, 128), jnp.float32),
    ],
)
def _sc_count(dst_hbm, znp_hbm, out_hbm, dsti_v, cpart_v):
    c = lax.axis_index("c")
    s = lax.axis_index("s")
    wid = c * NS + s
    pltpu.sync_copy(znp_hbm, cpart_v)
    ones16 = jnp.full((16,), 1.0, jnp.float32)

    def body(i, carry):
        base = wid * EW + i * K
        pltpu.sync_copy(dst_hbm.at[pl.ds(base, K)], dsti_v)
        for jj in range(K // 16):
            cidx = dsti_v[pl.ds(jj * 16, 16)]
            plsc.addupdate_scatter(cpart_v, [cidx >> 7, cidx & 127], ones16)
        return carry

    lax.fori_loop(0, IT, body, 0)
    pltpu.sync_copy(cpart_v, out_hbm.at[wid])


@functools.partial(
    pl.kernel,
    out_type=(jax.ShapeDtypeStruct((NC, N_P, D), jnp.float32),
              jax.ShapeDtypeStruct((NW, NR, 128), jnp.float32)),
    mesh=_sc_mesh(),
    compiler_params=pltpu.CompilerParams(needs_layout_passes=False),
    scratch_types=[
        pltpu.VMEM((K,), jnp.int32),
        pltpu.VMEM((K,), jnp.int32),
        pltpu.VMEM((K, D), jnp.float32),
        pltpu.VMEM((NR, 128), jnp.float32),
        pltpu.VMEM((NR, 128), jnp.float32),
        pltpu.VMEM_SHARED((N_P, D), jnp.float32),
        pltpu.SemaphoreType.DMA,
    ],
)
def _sc_scatter_t(g_hbm, dis_hbm, src_hbm, dst_hbm, z128_hbm, znp_hbm,
                  pout, tout, srci_v, dsti_v, rows_v, dis_v, tpart_v, acc,
                  sem):
    c = lax.axis_index("c")
    s = lax.axis_index("s")
    wid = c * NS + s
    pltpu.sync_copy(z128_hbm, acc.at[pl.ds(s * RPT, RPT)])
    pltpu.sync_copy(znp_hbm, tpart_v)
    pltpu.sync_copy(dis_hbm, dis_v)
    plsc.subcore_barrier()

    def body(i, carry):
        base = wid * EW + i * K
        pltpu.sync_copy(src_hbm.at[pl.ds(base, K)], srci_v)
        pltpu.sync_copy(dst_hbm.at[pl.ds(base, K)], dsti_v)
        pltpu.async_copy(g_hbm.at[srci_v], rows_v, sem).wait()
        pltpu.sync_copy(rows_v, acc.at[dsti_v], add=True)
        for jj in range(K // 16):
            didx = dsti_v[pl.ds(jj * 16, 16)]
            sidx = srci_v[pl.ds(jj * 16, 16)]
            dvals = plsc.load_gather(dis_v, [didx >> 7, didx & 127])
            plsc.addupdate_scatter(tpart_v, [sidx >> 7, sidx & 127], dvals)
        return carry

    lax.fori_loop(0, IT, body, 0)
    plsc.subcore_barrier()
    pltpu.sync_copy(acc.at[pl.ds(s * RPT, RPT)],
                    pout.at[c, pl.ds(s * RPT, RPT)])
    pltpu.sync_copy(tpart_v, tout.at[wid])


@functools.partial(
    pl.kernel,
    out_type=jax.ShapeDtypeStruct((NC, N_P, D), jnp.float32),
    mesh=_sc_mesh(),
    compiler_params=pltpu.CompilerParams(needs_layout_passes=False),
    scratch_types=[
        pltpu.VMEM((K,), jnp.int32),
        pltpu.VMEM((K,), jnp.int32),
        pltpu.VMEM((K, D), jnp.float32),
        pltpu.VMEM_SHARED((N_P, D), jnp.float32),
        pltpu.SemaphoreType.DMA,
    ],
)
def _sc_scatter(g_hbm, src_hbm, dst_hbm, z128_hbm, pout,
                srci_v, dsti_v, rows_v, acc, sem):
    c = lax.axis_index("c")
    s = lax.axis_index("s")
    wid = c * NS + s
    pltpu.sync_copy(z128_hbm, acc.at[pl.ds(s * RPT, RPT)])
    plsc.subcore_barrier()

    def body(i, carry):
        base = wid * EW + i * K
        pltpu.sync_copy(src_hbm.at[pl.ds(base, K)], srci_v)
        pltpu.sync_copy(dst_hbm.at[pl.ds(base, K)], dsti_v)
        pltpu.async_copy(g_hbm.at[srci_v], rows_v, sem).wait()
        pltpu.sync_copy(rows_v, acc.at[dsti_v], add=True)
        return carry

    lax.fori_loop(0, IT, body, 0)
    plsc.subcore_barrier()
    pltpu.sync_copy(acc.at[pl.ds(s * RPT, RPT)],
                    pout.at[c, pl.ds(s * RPT, RPT)])


def _tc1_body(x_ref, cnt_ref, w_ref, dis_ref, g_ref):
    ones_nw = jnp.ones((NW, 1), jnp.float32)
    cnt = lax.dot_general(cnt_ref[...], ones_nw, (((0,), (0,)), ((), ())),
                          preferred_element_type=jnp.float32)
    dis = lax.rsqrt(1.0 + cnt)
    dis_ref[...] = dis
    g_ref[...] = dis * jnp.dot(x_ref[...], w_ref[...],
                               preferred_element_type=jnp.float32)


def _tc1(x_p, cnt, W1):
    return pl.pallas_call(
        _tc1_body,
        grid=(TGRID,),
        in_specs=[
            pl.BlockSpec((RB, D), lambda i: (i, 0)),
            pl.BlockSpec((NW, RB), lambda i: (0, i)),
            pl.BlockSpec((D, D), lambda i: (0, 0)),
        ],
        out_specs=[
            pl.BlockSpec((RB, 1), lambda i: (i, 0)),
            pl.BlockSpec((RB, D), lambda i: (i, 0)),
        ],
        out_shape=[
            jax.ShapeDtypeStruct((N_P, 1), jnp.float32),
            jax.ShapeDtypeStruct((N_P, D), jnp.float32),
        ],
    )(x_p, cnt, W1)


def _tc2_body(p_ref, g_ref, dis_ref, b_ref, w_ref, t_ref, gout_ref,
              tsum_ref):
    dis = dis_ref[...]
    h = dis * (p_ref[0] + p_ref[1] + g_ref[...]) + b_ref[...]
    h = jnp.maximum(h, 0.0)
    gout_ref[...] = dis * jnp.dot(h, w_ref[...],
                                  preferred_element_type=jnp.float32)
    ones_nw = jnp.ones((NW, 1), jnp.float32)
    tsum_ref[...] = lax.dot_general(
        t_ref[...], ones_nw, (((0,), (0,)), ((), ())),
        preferred_element_type=jnp.float32)


def _tc2(p1, g1, dis, b1r, W2, tpart):
    return pl.pallas_call(
        _tc2_body,
        grid=(TGRID,),
        in_specs=[
            pl.BlockSpec((NC, RB, D), lambda i: (0, i, 0)),
            pl.BlockSpec((RB, D), lambda i: (i, 0)),
            pl.BlockSpec((RB, 1), lambda i: (i, 0)),
            pl.BlockSpec((1, D), lambda i: (0, 0)),
            pl.BlockSpec((D, D), lambda i: (0, 0)),
            pl.BlockSpec((NW, RB), lambda i: (0, i)),
        ],
        out_specs=[
            pl.BlockSpec((RB, D), lambda i: (i, 0)),
            pl.BlockSpec((RB, 1), lambda i: (i, 0)),
        ],
        out_shape=[
            jax.ShapeDtypeStruct((N_P, D), jnp.float32),
            jax.ShapeDtypeStruct((N_P, 1), jnp.float32),
        ],
    )(p1, g1, dis, b1r, W2, tpart)


def _tc3_body(p_ref, g_ref, dis_ref, b2_ref, t_ref, w3_ref, b3_ref,
              wl_ref, bl_ref, out_ref, racc):
    i = pl.program_id(0)
    dis = dis_ref[...]
    h2 = dis * (p_ref[0] + p_ref[1] + g_ref[...]) + b2_ref[...]
    h2 = jnp.maximum(h2, 0.0)
    q = dis * (t_ref[...] + dis)
    row = i * RB + lax.broadcasted_iota(jnp.int32, (RB, 1), 0)
    q = jnp.where(row < N, q, 0.0)
    contrib = jnp.sum(q * h2, axis=0, keepdims=True)

    @pl.when(i == 0)
    def _():
        racc[...] = contrib

    @pl.when(i > 0)
    def _():
        racc[...] = racc[...] + contrib

    @pl.when(i == TGRID - 1)
    def _():
        pooled = jnp.dot(racc[...] * (1.0 / N), w3_ref[...],
                         preferred_element_type=jnp.float32) + b3_ref[...]
        out_ref[...] = jnp.dot(pooled, wl_ref[...],
                               preferred_element_type=jnp.float32) + bl_ref[...]


def _tc3(p2, g2, dis, b2r, t, W3, b3r, Wl, blr):
    return pl.pallas_call(
        _tc3_body,
        grid=(TGRID,),
        in_specs=[
            pl.BlockSpec((NC, RB, D), lambda i: (0, i, 0)),
            pl.BlockSpec((RB, D), lambda i: (i, 0)),
            pl.BlockSpec((RB, 1), lambda i: (i, 0)),
            pl.BlockSpec((1, D), lambda i: (0, 0)),
            pl.BlockSpec((RB, 1), lambda i: (i, 0)),
            pl.BlockSpec((D, D), lambda i: (0, 0)),
            pl.BlockSpec((1, D), lambda i: (0, 0)),
            pl.BlockSpec((D, C), lambda i: (0, 0)),
            pl.BlockSpec((1, C), lambda i: (0, 0)),
        ],
        out_specs=pl.BlockSpec((1, C), lambda i: (0, 0)),
        out_shape=jax.ShapeDtypeStruct((1, C), jnp.float32),
        scratch_shapes=[pltpu.VMEM((1, D), jnp.float32)],
    )(p2, g2, dis, b2r, t, W3, b3r, Wl, blr)


def kernel(x, edge_index, W1, b1, W2, b2, W3, b3, Wl, bl):
    pad = jnp.full((E_P - E,), N, dtype=jnp.int32)
    src_p = jnp.concatenate([edge_index[0], pad])
    dst_p = jnp.concatenate([edge_index[1], pad])
    x_p = jnp.pad(x, ((0, N_P - N), (0, 0)))
    z128 = jnp.zeros((RPT, D), jnp.float32)
    znp = jnp.zeros((NR, 128), jnp.float32)
    b1r = b1.reshape(1, D)
    b2r = b2.reshape(1, D)
    b3r = b3.reshape(1, D)
    blr = bl.reshape(1, C)

    cnt = _sc_count(dst_p, znp)
    dis, g1 = _tc1(x_p, cnt.reshape(NW, N_P), W1)
    p1, tpart = _sc_scatter_t(g1, dis.reshape(NR, 128), src_p, dst_p,
                              z128, znp)
    g2, tsum = _tc2(p1, g1, dis, b1r, W2, tpart.reshape(NW, N_P))
    p2 = _sc_scatter(g2, src_p, dst_p, z128)
    return _tc3(p2, g2, dis, b2r, tsum, W3, b3r, Wl, blr)
